# Initial kernel scaffold; baseline (speedup 1.0000x reference)
import jax
import jax.numpy as jnp
from jax.experimental import pallas as pl


def kernel(feat, pos, senders, receivers, edge_attribute, W_e1, b_e1, W_e2, b_e2, W_n1, b_n1, W_n2, b_n2, W_p1, b_p1, W_p2):
    raise NotImplementedError("write your pallas kernel here")


# speedup vs baseline: 3.7998x; 3.7998x over previous
import functools

import jax
import jax.numpy as jnp
from jax import lax
from jax.experimental import pallas as pl
from jax.experimental.pallas import tpu as pltpu
from jax.experimental.pallas import tpu_sc as plsc

N = 10000
E = 320000
EP = 327680
D = 128
H = 128
DE = 16

NC = 2
NS = 16
NW = NC * NS
C = 128
NCHUNK = E // C

BN = 1000
BN1 = 1024
NP = 10240
BE = 2048

SLAB = 624
PSH = 30720
PSLAB = 1920

F32 = jnp.float32


def _silu(x):
    return x * jax.nn.sigmoid(x)


def _scmesh():
    return plsc.VectorSubcoreMesh(core_axis_name="c", subcore_axis_name="s",
                                  num_cores=NC, num_subcores=NS)


def _table_body(feat_ref, pos_ref, ws_ref, wr_ref, be1_ref, wrad_ref,
                ts_ref, tr_ref):
    f = feat_ref[...]
    p = pos_ref[...]
    n2 = jnp.sum(p * p, axis=1, keepdims=True) * wrad_ref[...]
    ts_ref[...] = jnp.dot(f, ws_ref[...], preferred_element_type=F32) + n2
    tr_ref[...] = (jnp.dot(f, wr_ref[...], preferred_element_type=F32)
                   + n2 + be1_ref[...])


def _build_tables(feat, pos, w_s, w_r, b_e1, w_rad):
    return pl.pallas_call(
        _table_body,
        grid=(N // BN,),
        in_specs=[
            pl.BlockSpec((BN, D), lambda i: (i, 0)),
            pl.BlockSpec((BN, 3), lambda i: (i, 0)),
            pl.BlockSpec((D, D), lambda i: (0, 0)),
            pl.BlockSpec((D, D), lambda i: (0, 0)),
            pl.BlockSpec((1, D), lambda i: (0, 0)),
            pl.BlockSpec((1, D), lambda i: (0, 0)),
        ],
        out_specs=[
            pl.BlockSpec((BN, D), lambda i: (i, 0)),
            pl.BlockSpec((BN, D), lambda i: (i, 0)),
        ],
        out_shape=[
            jax.ShapeDtypeStruct((N, D), F32),
            jax.ShapeDtypeStruct((N, D), F32),
        ],
    )(feat, pos, w_s, w_r, b_e1, w_rad)


def _gather_body(ts_hbm, tr_hbm, s_hbm, r_hbm, px_hbm, py_hbm, pz_hbm,
                 abs_out, abr_out, dot_out, cdp_out,
                 sidx, ridx, rows_s, rows_r,
                 psx, psy, psz, prx, pry, prz, smallb,
                 sem_s, sem_r, sem_p):
    wid = lax.axis_index("s") * NC + lax.axis_index("c")
    ntrips = (NCHUNK - wid + NW - 1) // NW

    def body(k, _):
        c = wid + k * NW
        base = pl.multiple_of(c * C, C)
        pltpu.sync_copy(s_hbm.at[pl.ds(base, C)], sidx)
        pltpu.sync_copy(r_hbm.at[pl.ds(base, C)], ridx)
        cp_s = pltpu.async_copy(ts_hbm.at[sidx], rows_s, sem_s)
        cp_r = pltpu.async_copy(tr_hbm.at[ridx], rows_r, sem_r)
        gs = [pltpu.async_copy(px_hbm.at[sidx], psx, sem_p),
              pltpu.async_copy(py_hbm.at[sidx], psy, sem_p),
              pltpu.async_copy(pz_hbm.at[sidx], psz, sem_p),
              pltpu.async_copy(px_hbm.at[ridx], prx, sem_p),
              pltpu.async_copy(py_hbm.at[ridx], pry, sem_p),
              pltpu.async_copy(pz_hbm.at[ridx], prz, sem_p)]
        for g in gs:
            g.wait()

        for g in range(C // 16):
            sl = pl.ds(g * 16, 16)
            sx, sy, sz = psx[sl], psy[sl], psz[sl]
            rx, ry, rz = prx[sl], pry[sl], prz[sl]
            smallb[sl] = sx * rx + sy * ry + sz * rz
            smallb[pl.ds(C + g * 16, 16)] = sx - rx
            smallb[pl.ds(2 * C + g * 16, 16)] = sy - ry
            smallb[pl.ds(3 * C + g * 16, 16)] = sz - rz

        pltpu.sync_copy(smallb.at[pl.ds(0, C)], dot_out.at[pl.ds(base, C)])
        cbase = pl.multiple_of(c * (3 * C), C)
        pltpu.sync_copy(smallb.at[pl.ds(C, 3 * C)],
                        cdp_out.at[pl.ds(cbase, 3 * C)])
        cp_s.wait()
        cp_r.wait()
        pltpu.sync_copy(rows_s, abs_out.at[pl.ds(base, C)])
        pltpu.sync_copy(rows_r, abr_out.at[pl.ds(base, C)])
        return 0

    lax.fori_loop(0, ntrips, body, 0)


def _gather_edges(ts, tr, senders, receivers, px, py, pz):
    fn = functools.partial(
        pl.kernel,
        out_type=[
            jax.ShapeDtypeStruct((EP, D), F32),
            jax.ShapeDtypeStruct((EP, D), F32),
            jax.ShapeDtypeStruct((EP,), F32),
            jax.ShapeDtypeStruct((3 * E,), F32),
        ],
        mesh=_scmesh(),
        scratch_types=[
            pltpu.VMEM((C,), jnp.int32),
            pltpu.VMEM((C,), jnp.int32),
            pltpu.VMEM((C, D), F32),
            pltpu.VMEM((C, D), F32),
            pltpu.VMEM((C,), F32),
            pltpu.VMEM((C,), F32),
            pltpu.VMEM((C,), F32),
            pltpu.VMEM((C,), F32),
            pltpu.VMEM((C,), F32),
            pltpu.VMEM((C,), F32),
            pltpu.VMEM((4 * C,), F32),
            pltpu.SemaphoreType.DMA,
            pltpu.SemaphoreType.DMA,
            pltpu.SemaphoreType.DMA,
        ],
    )(_gather_body)
    return fn(ts, tr, senders, receivers, px, py, pz)


def _edge_body(abs_ref, abr_ref, attr_ref, dot_ref, wa_ref, wrad2_ref,
               we2_ref, be2_ref, wp1_ref, bp1_ref, wp2_ref,
               msg_ref, pc_ref):
    dcol = dot_ref[...].reshape((BE, 1))
    pre1 = (abs_ref[...] + abr_ref[...]
            + jnp.dot(attr_ref[...], wa_ref[...], preferred_element_type=F32)
            + dcol * wrad2_ref[...])
    h = _silu(pre1)
    m = _silu(jnp.dot(h, we2_ref[...], preferred_element_type=F32) + be2_ref[...])
    msg_ref[...] = m
    h2 = _silu(jnp.dot(m, wp1_ref[...], preferred_element_type=F32) + bp1_ref[...])
    pc_ref[...] = jnp.sum(h2 * wp2_ref[...], axis=1)


def _edge_mlp(ab_s, ab_r, attr, dot, w_a, w_rad2, w_e2, b_e2, w_p1, b_p1, w_p2row):
    return pl.pallas_call(
        _edge_body,
        grid=(EP // BE,),
        in_specs=[
            pl.BlockSpec((BE, D), lambda i: (i, 0)),
            pl.BlockSpec((BE, D), lambda i: (i, 0)),
            pl.BlockSpec((BE, DE), lambda i: (i, 0)),
            pl.BlockSpec((BE,), lambda i: (i,)),
            pl.BlockSpec((DE, D), lambda i: (0, 0)),
            pl.BlockSpec((1, D), lambda i: (0, 0)),
            pl.BlockSpec((D, D), lambda i: (0, 0)),
            pl.BlockSpec((1, D), lambda i: (0, 0)),
            pl.BlockSpec((D, D), lambda i: (0, 0)),
            pl.BlockSpec((1, D), lambda i: (0, 0)),
            pl.BlockSpec((1, D), lambda i: (0, 0)),
        ],
        out_specs=[
            pl.BlockSpec((BE, D), lambda i: (i, 0)),
            pl.BlockSpec((BE,), lambda i: (i,)),
        ],
        out_shape=[
            jax.ShapeDtypeStruct((EP, D), F32),
            jax.ShapeDtypeStruct((EP,), F32),
        ],
    )(ab_s, ab_r, attr, dot, w_a, w_rad2, w_e2, b_e2, w_p1, b_p1, w_p2row)


def _scatter_body(msg_hbm, pc_hbm, cdp_hbm, r_hbm, s_hbm,
                  agg_out, pos_out,
                  ridx, sidx, mbuf, pcb, cdb, tbx, tby, tbz, bounce,
                  agg_sh, pos_shx, pos_shy, pos_shz):
    cid = lax.axis_index("c")
    sid = lax.axis_index("s")
    wid = sid * NC + cid
    z16 = jnp.zeros((16,), F32)
    planes = (pos_shx, pos_shy, pos_shz)

    def zrow(j, _):
        for q in range(D // 16):
            mbuf[j, pl.ds(q * 16, 16)] = z16
        return 0

    lax.fori_loop(0, C, zrow, 0)
    for g in range(SLAB // 16):
        bounce[pl.ds(g * 16, 16)] = z16

    slab0 = pl.multiple_of(sid * SLAB, 8)

    def zslab(j, _):
        off = pl.multiple_of(slab0 + j * C, 8)
        pltpu.sync_copy(mbuf, agg_sh.at[pl.ds(off, C)])
        return 0

    lax.fori_loop(0, SLAB // C, zslab, 0)
    off = pl.multiple_of(slab0 + (SLAB // C) * C, 8)
    pltpu.sync_copy(mbuf.at[pl.ds(0, SLAB % C)], agg_sh.at[pl.ds(off, SLAB % C)])
    for plane in planes:
        pltpu.sync_copy(bounce, plane.at[pl.ds(slab0, SLAB)])

    @pl.when(sid == NS - 1)
    def _():
        tail = pl.multiple_of(NS * SLAB, 8)
        rem = N - NS * SLAB
        pltpu.sync_copy(mbuf.at[pl.ds(0, rem)], agg_sh.at[pl.ds(tail, rem)])
        for plane in planes:
            pltpu.sync_copy(bounce.at[pl.ds(0, rem)], plane.at[pl.ds(tail, rem)])

    plsc.subcore_barrier()

    ntrips = (NCHUNK - wid + NW - 1) // NW

    def body(k, _):
        c = wid + k * NW
        base = pl.multiple_of(c * C, C)
        pltpu.sync_copy(r_hbm.at[pl.ds(base, C)], ridx)
        pltpu.sync_copy(msg_hbm.at[pl.ds(base, C)], mbuf)
        pltpu.sync_copy(mbuf, agg_sh.at[ridx], add=True)

        pltpu.sync_copy(s_hbm.at[pl.ds(base, C)], sidx)
        pltpu.sync_copy(pc_hbm.at[pl.ds(base, C)], pcb)
        cbase = pl.multiple_of(c * (3 * C), C)
        pltpu.sync_copy(cdp_hbm.at[pl.ds(cbase, 3 * C)], cdb)
        for g in range(C // 16):
            sl = pl.ds(g * 16, 16)
            pcv = pcb[sl]
            tbx[sl] = jnp.clip(cdb[sl] * pcv, -100.0, 100.0)
            tby[sl] = jnp.clip(cdb[pl.ds(C + g * 16, 16)] * pcv, -100.0, 100.0)
            tbz[sl] = jnp.clip(cdb[pl.ds(2 * C + g * 16, 16)] * pcv, -100.0, 100.0)
        pltpu.sync_copy(tbx, pos_shx.at[sidx], add=True)
        pltpu.sync_copy(tby, pos_shy.at[sidx], add=True)
        pltpu.sync_copy(tbz, pos_shz.at[sidx], add=True)
        return 0

    lax.fori_loop(0, ntrips, body, 0)
    plsc.subcore_barrier()

    pltpu.sync_copy(agg_sh.at[pl.ds(slab0, SLAB)],
                    agg_out.at[cid, pl.ds(slab0, SLAB)])
    for comp, plane in enumerate(planes):
        dst = pl.multiple_of((cid * 3 + comp) * NP + slab0, 8)
        pltpu.sync_copy(plane.at[pl.ds(slab0, SLAB)], bounce)
        pltpu.sync_copy(bounce, pos_out.at[pl.ds(dst, SLAB)])

    @pl.when(sid == NS - 1)
    def _():
        t0 = pl.multiple_of(NS * SLAB, 8)
        rem = N - NS * SLAB
        pltpu.sync_copy(agg_sh.at[pl.ds(t0, rem)],
                        agg_out.at[cid, pl.ds(t0, rem)])
        for comp, plane in enumerate(planes):
            dst = pl.multiple_of((cid * 3 + comp) * NP + t0, 8)
            pltpu.sync_copy(plane.at[pl.ds(t0, rem)], bounce.at[pl.ds(0, rem)])
            pltpu.sync_copy(bounce.at[pl.ds(0, rem)], pos_out.at[pl.ds(dst, rem)])


def _scatter_edges(msg, pc, cdp, receivers, senders):
    fn = functools.partial(
        pl.kernel,
        out_type=[
            jax.ShapeDtypeStruct((NC, N, D), F32),
            jax.ShapeDtypeStruct((NC * 3 * NP,), F32),
        ],
        mesh=_scmesh(),
        scratch_types=[
            pltpu.VMEM((C,), jnp.int32),
            pltpu.VMEM((C,), jnp.int32),
            pltpu.VMEM((C, D), F32),
            pltpu.VMEM((C,), F32),
            pltpu.VMEM((3 * C,), F32),
            pltpu.VMEM((C,), F32),
            pltpu.VMEM((C,), F32),
            pltpu.VMEM((C,), F32),
            pltpu.VMEM((SLAB,), F32),
            pltpu.VMEM_SHARED((N, D), F32),
            pltpu.VMEM_SHARED((N,), F32),
            pltpu.VMEM_SHARED((N,), F32),
            pltpu.VMEM_SHARED((N,), F32),
        ],
    )(_scatter_body)
    return fn(msg, pc, cdp, receivers, senders)


def _node_body(feat_ref, aggp_ref, px_ref, py_ref, pz_ref,
               p0x_ref, p0y_ref, p0z_ref, p1x_ref, p1y_ref, p1z_ref,
               wn1a_ref, wn1b_ref, bn1_ref, wn2_ref, bn2_ref,
               fout_ref, pxo_ref, pyo_ref, pzo_ref):
    f = feat_ref[...]
    agg = aggp_ref[0] + aggp_ref[1]
    x = _silu(jnp.dot(f, wn1a_ref[...], preferred_element_type=F32)
              + jnp.dot(agg, wn1b_ref[...], preferred_element_type=F32)
              + bn1_ref[...])
    fout_ref[...] = f + jnp.dot(x, wn2_ref[...], preferred_element_type=F32) + bn2_ref[...]
    pxo_ref[...] = px_ref[...] + p0x_ref[...] + p1x_ref[...]
    pyo_ref[...] = py_ref[...] + p0y_ref[...] + p1y_ref[...]
    pzo_ref[...] = pz_ref[...] + p0z_ref[...] + p1z_ref[...]


def _node_update(feat, aggp, posacc, pxp, pyp, pzp,
                 w_n1a, w_n1b, b_n1, w_n2, b_n2):
    nb = NP // BN1
    acc_spec = lambda p: pl.BlockSpec((BN1,), lambda i, p=p: (i + p * nb,))
    return pl.pallas_call(
        _node_body,
        grid=(N // BN,),
        in_specs=[
            pl.BlockSpec((BN, D), lambda i: (i, 0)),
            pl.BlockSpec((NC, BN, D), lambda i: (0, i, 0)),
            pl.BlockSpec((BN1,), lambda i: (i,)),
            pl.BlockSpec((BN1,), lambda i: (i,)),
            pl.BlockSpec((BN1,), lambda i: (i,)),
            acc_spec(0), acc_spec(1), acc_spec(2),
            acc_spec(3), acc_spec(4), acc_spec(5),
            pl.BlockSpec((D, D), lambda i: (0, 0)),
            pl.BlockSpec((D, D), lambda i: (0, 0)),
            pl.BlockSpec((1, D), lambda i: (0, 0)),
            pl.BlockSpec((D, D), lambda i: (0, 0)),
            pl.BlockSpec((1, D), lambda i: (0, 0)),
        ],
        out_specs=[
            pl.BlockSpec((BN, D), lambda i: (i, 0)),
            pl.BlockSpec((BN1,), lambda i: (i,)),
            pl.BlockSpec((BN1,), lambda i: (i,)),
            pl.BlockSpec((BN1,), lambda i: (i,)),
        ],
        out_shape=[
            jax.ShapeDtypeStruct((N, D), F32),
            jax.ShapeDtypeStruct((NP,), F32),
            jax.ShapeDtypeStruct((NP,), F32),
            jax.ShapeDtypeStruct((NP,), F32),
        ],
    )(feat, aggp, pxp, pyp, pzp,
      posacc, posacc, posacc, posacc, posacc, posacc,
      w_n1a, w_n1b, b_n1, w_n2, b_n2)


def kernel(feat, pos, senders, receivers, edge_attribute,
           W_e1, b_e1, W_e2, b_e2, W_n1, b_n1, W_n2, b_n2, W_p1, b_p1, W_p2):
    senders = senders.astype(jnp.int32)
    receivers = receivers.astype(jnp.int32)

    w_s = W_e1[:D]
    w_r = W_e1[D:2 * D]
    w_rad = W_e1[2 * D:2 * D + 1]
    w_rad2 = -2.0 * w_rad
    w_a = W_e1[2 * D + 1:]

    px = pos[:, 0]
    py = pos[:, 1]
    pz = pos[:, 2]
    pxp = jnp.pad(px, (0, NP - N))
    pyp = jnp.pad(py, (0, NP - N))
    pzp = jnp.pad(pz, (0, NP - N))
    attr_p = jnp.pad(edge_attribute, ((0, EP - E), (0, 0)))

    ts, tr = _build_tables(feat, pos, w_s, w_r, b_e1.reshape(1, D), w_rad)
    ab_s, ab_r, dot, cdp = _gather_edges(ts, tr, senders, receivers, px, py, pz)
    msg, pc = _edge_mlp(ab_s, ab_r, attr_p, dot, w_a, w_rad2,
                        W_e2, b_e2.reshape(1, D), W_p1, b_p1.reshape(1, D),
                        W_p2.reshape(1, D))
    aggp, posacc = _scatter_edges(msg, pc, cdp, receivers, senders)
    feat_new, pxo, pyo, pzo = _node_update(feat, aggp, posacc, pxp, pyp, pzp,
                                           W_n1[:D], W_n1[D:],
                                           b_n1.reshape(1, D), W_n2,
                                           b_n2.reshape(1, D))
    pos_new = jnp.stack([pxo[:N], pyo[:N], pzo[:N]], axis=1)
    return feat_new, pos_new


# speedup vs baseline: 4.3016x; 1.1320x over previous
import functools

import jax
import jax.numpy as jnp
from jax import lax
from jax.experimental import pallas as pl
from jax.experimental.pallas import tpu as pltpu
from jax.experimental.pallas import tpu_sc as plsc

N = 10000
E = 320000
EP = 327680
D = 128
H = 128
DE = 16

NC = 2
NS = 16
NW = NC * NS
C = 128
NCHUNK = E // C

BN = 1000
BN1 = 1024
NP = 10240
BE = 2048

SLAB = 624
PSH = 30720
PSLAB = 1920

F32 = jnp.float32


def _silu(x):
    return x * jax.nn.sigmoid(x)


def _scmesh():
    return plsc.VectorSubcoreMesh(core_axis_name="c", subcore_axis_name="s",
                                  num_cores=NC, num_subcores=NS)


def _table_body(feat_ref, pos_ref, ws_ref, wr_ref, be1_ref, wrad_ref,
                ts_ref, tr_ref):
    f = feat_ref[...]
    p = pos_ref[...]
    n2 = jnp.sum(p * p, axis=1, keepdims=True) * wrad_ref[...]
    ts_ref[...] = jnp.dot(f, ws_ref[...], preferred_element_type=F32) + n2
    tr_ref[...] = (jnp.dot(f, wr_ref[...], preferred_element_type=F32)
                   + n2 + be1_ref[...])


def _build_tables(feat, pos, w_s, w_r, b_e1, w_rad):
    return pl.pallas_call(
        _table_body,
        grid=(N // BN,),
        in_specs=[
            pl.BlockSpec((BN, D), lambda i: (i, 0)),
            pl.BlockSpec((BN, 3), lambda i: (i, 0)),
            pl.BlockSpec((D, D), lambda i: (0, 0)),
            pl.BlockSpec((D, D), lambda i: (0, 0)),
            pl.BlockSpec((1, D), lambda i: (0, 0)),
            pl.BlockSpec((1, D), lambda i: (0, 0)),
        ],
        out_specs=[
            pl.BlockSpec((BN, D), lambda i: (i, 0)),
            pl.BlockSpec((BN, D), lambda i: (i, 0)),
        ],
        out_shape=[
            jax.ShapeDtypeStruct((N, D), F32),
            jax.ShapeDtypeStruct((N, D), F32),
        ],
    )(feat, pos, w_s, w_r, b_e1, w_rad)


def _gather_body(ts_hbm, tr_hbm, s_hbm, r_hbm, px_hbm, py_hbm, pz_hbm,
                 abs_out, abr_out, dot_out, cdp_out,
                 sidx0, ridx0, rows_s0, rows_r0,
                 psx0, psy0, psz0, prx0, pry0, prz0, smallb0,
                 sidx1, ridx1, rows_s1, rows_r1,
                 psx1, psy1, psz1, prx1, pry1, prz1, smallb1,
                 bounce, shx, shy, shz,
                 sem_s0, sem_r0, sem_p0, sem_s1, sem_r1, sem_p1):
    cid = lax.axis_index("c")
    sid = lax.axis_index("s")
    wid = sid * NC + cid
    nt = (NCHUNK - wid + NW - 1) // NW
    sets = [
        (sidx0, ridx0, rows_s0, rows_r0, psx0, psy0, psz0, prx0, pry0, prz0,
         smallb0, sem_s0, sem_r0, sem_p0),
        (sidx1, ridx1, rows_s1, rows_r1, psx1, psy1, psz1, prx1, pry1, prz1,
         smallb1, sem_s1, sem_r1, sem_p1),
    ]

    off = pl.multiple_of(sid * SLAB, 8)
    for plane_hbm, sh in ((px_hbm, shx), (py_hbm, shy), (pz_hbm, shz)):
        pltpu.sync_copy(plane_hbm.at[pl.ds(off, SLAB)], bounce)
        pltpu.sync_copy(bounce, sh.at[pl.ds(off, SLAB)])

    @pl.when(sid == NS - 1)
    def _():
        t0 = pl.multiple_of(NS * SLAB, 8)
        rem = N - NS * SLAB
        for plane_hbm, sh in ((px_hbm, shx), (py_hbm, shy), (pz_hbm, shz)):
            pltpu.sync_copy(plane_hbm.at[pl.ds(t0, rem)],
                            bounce.at[pl.ds(0, rem)])
            pltpu.sync_copy(bounce.at[pl.ds(0, rem)], sh.at[pl.ds(t0, rem)])

    plsc.subcore_barrier()

    def start(st, c):
        (sidx, ridx, rows_s, rows_r, psx, psy, psz, prx, pry, prz,
         _sm, sem_s, sem_r, sem_p) = st
        base = pl.multiple_of(c * C, C)
        pltpu.sync_copy(s_hbm.at[pl.ds(base, C)], sidx)
        pltpu.sync_copy(r_hbm.at[pl.ds(base, C)], ridx)
        pltpu.async_copy(ts_hbm.at[sidx], rows_s, sem_s)
        pltpu.async_copy(tr_hbm.at[ridx], rows_r, sem_r)
        pltpu.async_copy(shx.at[sidx], psx, sem_p)
        pltpu.async_copy(shy.at[sidx], psy, sem_p)
        pltpu.async_copy(shz.at[sidx], psz, sem_p)
        pltpu.async_copy(shx.at[ridx], prx, sem_p)
        pltpu.async_copy(shy.at[ridx], pry, sem_p)
        pltpu.async_copy(shz.at[ridx], prz, sem_p)

    def process(st, c):
        (sidx, ridx, rows_s, rows_r, psx, psy, psz, prx, pry, prz,
         smallb, sem_s, sem_r, sem_p) = st
        base = pl.multiple_of(c * C, C)
        for buf in (psx, psy, psz, prx, pry, prz):
            pltpu.make_async_copy(px_hbm.at[sidx], buf, sem_p).wait()

        for g in range(C // 16):
            sl = pl.ds(g * 16, 16)
            sx, sy, sz = psx[sl], psy[sl], psz[sl]
            rx, ry, rz = prx[sl], pry[sl], prz[sl]
            smallb[sl] = sx * rx + sy * ry + sz * rz
            smallb[pl.ds(C + g * 16, 16)] = sx - rx
            smallb[pl.ds(2 * C + g * 16, 16)] = sy - ry
            smallb[pl.ds(3 * C + g * 16, 16)] = sz - rz

        pltpu.sync_copy(smallb.at[pl.ds(0, C)], dot_out.at[pl.ds(base, C)])
        cbase = pl.multiple_of(c * (3 * C), C)
        pltpu.sync_copy(smallb.at[pl.ds(C, 3 * C)],
                        cdp_out.at[pl.ds(cbase, 3 * C)])
        pltpu.make_async_copy(ts_hbm.at[sidx], rows_s, sem_s).wait()
        pltpu.make_async_copy(tr_hbm.at[ridx], rows_r, sem_r).wait()
        pltpu.sync_copy(rows_s, abs_out.at[pl.ds(base, C)])
        pltpu.sync_copy(rows_r, abr_out.at[pl.ds(base, C)])

    start(sets[0], wid)

    def pair(k2, _):
        for half in (0, 1):
            k = 2 * k2 + half

            @pl.when(k + 1 < nt)
            def _():
                start(sets[1 - half], wid + (k + 1) * NW)

            @pl.when(k < nt)
            def _():
                process(sets[half], wid + k * NW)
        return 0

    lax.fori_loop(0, 40, pair, 0)


def _gather_edges(ts, tr, senders, receivers, px, py, pz):
    fn = functools.partial(
        pl.kernel,
        out_type=[
            jax.ShapeDtypeStruct((EP, D), F32),
            jax.ShapeDtypeStruct((EP, D), F32),
            jax.ShapeDtypeStruct((EP,), F32),
            jax.ShapeDtypeStruct((3 * E,), F32),
        ],
        mesh=_scmesh(),
        scratch_types=(
            2 * ([pltpu.VMEM((C,), jnp.int32)] * 2
                 + [pltpu.VMEM((C, D), F32)] * 2
                 + [pltpu.VMEM((C,), F32)] * 6
                 + [pltpu.VMEM((4 * C,), F32)])
            + [pltpu.VMEM((SLAB,), F32)]
            + [pltpu.VMEM_SHARED((N,), F32)] * 3
            + [pltpu.SemaphoreType.DMA] * 6
        ),
    )(_gather_body)
    return fn(ts, tr, senders, receivers, px, py, pz)


def _edge_body(abs_ref, abr_ref, attr_ref, dot_ref, wa_ref, wrad2_ref,
               we2_ref, be2_ref, wp1_ref, bp1_ref, wp2_ref,
               msg_ref, pc_ref):
    dcol = dot_ref[...].reshape((BE, 1))
    pre1 = (abs_ref[...] + abr_ref[...]
            + jnp.dot(attr_ref[...], wa_ref[...], preferred_element_type=F32)
            + dcol * wrad2_ref[...])
    h = _silu(pre1)
    m = _silu(jnp.dot(h, we2_ref[...], preferred_element_type=F32) + be2_ref[...])
    msg_ref[...] = m
    h2 = _silu(jnp.dot(m, wp1_ref[...], preferred_element_type=F32) + bp1_ref[...])
    pc_ref[...] = jnp.sum(h2 * wp2_ref[...], axis=1)


def _edge_mlp(ab_s, ab_r, attr, dot, w_a, w_rad2, w_e2, b_e2, w_p1, b_p1, w_p2row):
    return pl.pallas_call(
        _edge_body,
        grid=(EP // BE,),
        in_specs=[
            pl.BlockSpec((BE, D), lambda i: (i, 0)),
            pl.BlockSpec((BE, D), lambda i: (i, 0)),
            pl.BlockSpec((BE, DE), lambda i: (i, 0)),
            pl.BlockSpec((BE,), lambda i: (i,)),
            pl.BlockSpec((DE, D), lambda i: (0, 0)),
            pl.BlockSpec((1, D), lambda i: (0, 0)),
            pl.BlockSpec((D, D), lambda i: (0, 0)),
            pl.BlockSpec((1, D), lambda i: (0, 0)),
            pl.BlockSpec((D, D), lambda i: (0, 0)),
            pl.BlockSpec((1, D), lambda i: (0, 0)),
            pl.BlockSpec((1, D), lambda i: (0, 0)),
        ],
        out_specs=[
            pl.BlockSpec((BE, D), lambda i: (i, 0)),
            pl.BlockSpec((BE,), lambda i: (i,)),
        ],
        out_shape=[
            jax.ShapeDtypeStruct((EP, D), F32),
            jax.ShapeDtypeStruct((EP,), F32),
        ],
    )(ab_s, ab_r, attr, dot, w_a, w_rad2, w_e2, b_e2, w_p1, b_p1, w_p2row)


def _scatter_body(msg_hbm, pc_hbm, cdp_hbm, r_hbm, s_hbm,
                  agg_out, pos_out,
                  ridx, sidx, mbuf, pcb, cdb, tbx, tby, tbz, bounce,
                  agg_sh, pos_shx, pos_shy, pos_shz):
    cid = lax.axis_index("c")
    sid = lax.axis_index("s")
    wid = sid * NC + cid
    z16 = jnp.zeros((16,), F32)
    planes = (pos_shx, pos_shy, pos_shz)

    def zrow(j, _):
        for q in range(D // 16):
            mbuf[j, pl.ds(q * 16, 16)] = z16
        return 0

    lax.fori_loop(0, C, zrow, 0)
    for g in range(SLAB // 16):
        bounce[pl.ds(g * 16, 16)] = z16

    slab0 = pl.multiple_of(sid * SLAB, 8)

    def zslab(j, _):
        off = pl.multiple_of(slab0 + j * C, 8)
        pltpu.sync_copy(mbuf, agg_sh.at[pl.ds(off, C)])
        return 0

    lax.fori_loop(0, SLAB // C, zslab, 0)
    off = pl.multiple_of(slab0 + (SLAB // C) * C, 8)
    pltpu.sync_copy(mbuf.at[pl.ds(0, SLAB % C)], agg_sh.at[pl.ds(off, SLAB % C)])
    for plane in planes:
        pltpu.sync_copy(bounce, plane.at[pl.ds(slab0, SLAB)])

    @pl.when(sid == NS - 1)
    def _():
        tail = pl.multiple_of(NS * SLAB, 8)
        rem = N - NS * SLAB
        pltpu.sync_copy(mbuf.at[pl.ds(0, rem)], agg_sh.at[pl.ds(tail, rem)])
        for plane in planes:
            pltpu.sync_copy(bounce.at[pl.ds(0, rem)], plane.at[pl.ds(tail, rem)])

    plsc.subcore_barrier()

    ntrips = (NCHUNK - wid + NW - 1) // NW

    def body(k, _):
        c = wid + k * NW
        base = pl.multiple_of(c * C, C)
        pltpu.sync_copy(r_hbm.at[pl.ds(base, C)], ridx)
        pltpu.sync_copy(msg_hbm.at[pl.ds(base, C)], mbuf)
        pltpu.sync_copy(mbuf, agg_sh.at[ridx], add=True)

        pltpu.sync_copy(s_hbm.at[pl.ds(base, C)], sidx)
        pltpu.sync_copy(pc_hbm.at[pl.ds(base, C)], pcb)
        cbase = pl.multiple_of(c * (3 * C), C)
        pltpu.sync_copy(cdp_hbm.at[pl.ds(cbase, 3 * C)], cdb)
        for g in range(C // 16):
            sl = pl.ds(g * 16, 16)
            pcv = pcb[sl]
            tbx[sl] = jnp.clip(cdb[sl] * pcv, -100.0, 100.0)
            tby[sl] = jnp.clip(cdb[pl.ds(C + g * 16, 16)] * pcv, -100.0, 100.0)
            tbz[sl] = jnp.clip(cdb[pl.ds(2 * C + g * 16, 16)] * pcv, -100.0, 100.0)
        pltpu.sync_copy(tbx, pos_shx.at[sidx], add=True)
        pltpu.sync_copy(tby, pos_shy.at[sidx], add=True)
        pltpu.sync_copy(tbz, pos_shz.at[sidx], add=True)
        return 0

    lax.fori_loop(0, ntrips, body, 0)
    plsc.subcore_barrier()

    pltpu.sync_copy(agg_sh.at[pl.ds(slab0, SLAB)],
                    agg_out.at[cid, pl.ds(slab0, SLAB)])
    for comp, plane in enumerate(planes):
        dst = pl.multiple_of((cid * 3 + comp) * NP + slab0, 8)
        pltpu.sync_copy(plane.at[pl.ds(slab0, SLAB)], bounce)
        pltpu.sync_copy(bounce, pos_out.at[pl.ds(dst, SLAB)])

    @pl.when(sid == NS - 1)
    def _():
        t0 = pl.multiple_of(NS * SLAB, 8)
        rem = N - NS * SLAB
        pltpu.sync_copy(agg_sh.at[pl.ds(t0, rem)],
                        agg_out.at[cid, pl.ds(t0, rem)])
        for comp, plane in enumerate(planes):
            dst = pl.multiple_of((cid * 3 + comp) * NP + t0, 8)
            pltpu.sync_copy(plane.at[pl.ds(t0, rem)], bounce.at[pl.ds(0, rem)])
            pltpu.sync_copy(bounce.at[pl.ds(0, rem)], pos_out.at[pl.ds(dst, rem)])


def _scatter_edges(msg, pc, cdp, receivers, senders):
    fn = functools.partial(
        pl.kernel,
        out_type=[
            jax.ShapeDtypeStruct((NC, N, D), F32),
            jax.ShapeDtypeStruct((NC * 3 * NP,), F32),
        ],
        mesh=_scmesh(),
        scratch_types=[
            pltpu.VMEM((C,), jnp.int32),
            pltpu.VMEM((C,), jnp.int32),
            pltpu.VMEM((C, D), F32),
            pltpu.VMEM((C,), F32),
            pltpu.VMEM((3 * C,), F32),
            pltpu.VMEM((C,), F32),
            pltpu.VMEM((C,), F32),
            pltpu.VMEM((C,), F32),
            pltpu.VMEM((SLAB,), F32),
            pltpu.VMEM_SHARED((N, D), F32),
            pltpu.VMEM_SHARED((N,), F32),
            pltpu.VMEM_SHARED((N,), F32),
            pltpu.VMEM_SHARED((N,), F32),
        ],
    )(_scatter_body)
    return fn(msg, pc, cdp, receivers, senders)


def _node_body(feat_ref, aggp_ref, px_ref, py_ref, pz_ref,
               p0x_ref, p0y_ref, p0z_ref, p1x_ref, p1y_ref, p1z_ref,
               wn1a_ref, wn1b_ref, bn1_ref, wn2_ref, bn2_ref,
               fout_ref, pxo_ref, pyo_ref, pzo_ref):
    f = feat_ref[...]
    agg = aggp_ref[0] + aggp_ref[1]
    x = _silu(jnp.dot(f, wn1a_ref[...], preferred_element_type=F32)
              + jnp.dot(agg, wn1b_ref[...], preferred_element_type=F32)
              + bn1_ref[...])
    fout_ref[...] = f + jnp.dot(x, wn2_ref[...], preferred_element_type=F32) + bn2_ref[...]
    pxo_ref[...] = px_ref[...] + p0x_ref[...] + p1x_ref[...]
    pyo_ref[...] = py_ref[...] + p0y_ref[...] + p1y_ref[...]
    pzo_ref[...] = pz_ref[...] + p0z_ref[...] + p1z_ref[...]


def _node_update(feat, aggp, posacc, pxp, pyp, pzp,
                 w_n1a, w_n1b, b_n1, w_n2, b_n2):
    nb = NP // BN1
    acc_spec = lambda p: pl.BlockSpec((BN1,), lambda i, p=p: (i + p * nb,))
    return pl.pallas_call(
        _node_body,
        grid=(N // BN,),
        in_specs=[
            pl.BlockSpec((BN, D), lambda i: (i, 0)),
            pl.BlockSpec((NC, BN, D), lambda i: (0, i, 0)),
            pl.BlockSpec((BN1,), lambda i: (i,)),
            pl.BlockSpec((BN1,), lambda i: (i,)),
            pl.BlockSpec((BN1,), lambda i: (i,)),
            acc_spec(0), acc_spec(1), acc_spec(2),
            acc_spec(3), acc_spec(4), acc_spec(5),
            pl.BlockSpec((D, D), lambda i: (0, 0)),
            pl.BlockSpec((D, D), lambda i: (0, 0)),
            pl.BlockSpec((1, D), lambda i: (0, 0)),
            pl.BlockSpec((D, D), lambda i: (0, 0)),
            pl.BlockSpec((1, D), lambda i: (0, 0)),
        ],
        out_specs=[
            pl.BlockSpec((BN, D), lambda i: (i, 0)),
            pl.BlockSpec((BN1,), lambda i: (i,)),
            pl.BlockSpec((BN1,), lambda i: (i,)),
            pl.BlockSpec((BN1,), lambda i: (i,)),
        ],
        out_shape=[
            jax.ShapeDtypeStruct((N, D), F32),
            jax.ShapeDtypeStruct((NP,), F32),
            jax.ShapeDtypeStruct((NP,), F32),
            jax.ShapeDtypeStruct((NP,), F32),
        ],
    )(feat, aggp, pxp, pyp, pzp,
      posacc, posacc, posacc, posacc, posacc, posacc,
      w_n1a, w_n1b, b_n1, w_n2, b_n2)


def kernel(feat, pos, senders, receivers, edge_attribute,
           W_e1, b_e1, W_e2, b_e2, W_n1, b_n1, W_n2, b_n2, W_p1, b_p1, W_p2):
    senders = senders.astype(jnp.int32)
    receivers = receivers.astype(jnp.int32)

    w_s = W_e1[:D]
    w_r = W_e1[D:2 * D]
    w_rad = W_e1[2 * D:2 * D + 1]
    w_rad2 = -2.0 * w_rad
    w_a = W_e1[2 * D + 1:]

    px = pos[:, 0]
    py = pos[:, 1]
    pz = pos[:, 2]
    pxp = jnp.pad(px, (0, NP - N))
    pyp = jnp.pad(py, (0, NP - N))
    pzp = jnp.pad(pz, (0, NP - N))
    attr_p = jnp.pad(edge_attribute, ((0, EP - E), (0, 0)))

    ts, tr = _build_tables(feat, pos, w_s, w_r, b_e1.reshape(1, D), w_rad)
    ab_s, ab_r, dot, cdp = _gather_edges(ts, tr, senders, receivers, px, py, pz)
    msg, pc = _edge_mlp(ab_s, ab_r, attr_p, dot, w_a, w_rad2,
                        W_e2, b_e2.reshape(1, D), W_p1, b_p1.reshape(1, D),
                        W_p2.reshape(1, D))
    aggp, posacc = _scatter_edges(msg, pc, cdp, receivers, senders)
    feat_new, pxo, pyo, pzo = _node_update(feat, aggp, posacc, pxp, pyp, pzp,
                                           W_n1[:D], W_n1[D:],
                                           b_n1.reshape(1, D), W_n2,
                                           b_n2.reshape(1, D))
    pos_new = jnp.stack([pxo[:N], pyo[:N], pzo[:N]], axis=1)
    return feat_new, pos_new


# speedup vs baseline: 5.3551x; 1.2449x over previous
import functools

import jax
import jax.numpy as jnp
from jax import lax
from jax.experimental import pallas as pl
from jax.experimental.pallas import tpu as pltpu
from jax.experimental.pallas import tpu_sc as plsc

N = 10000
E = 320000
EP = 327680
D = 128
H = 128
DE = 16

NC = 2
NS = 16
NW = NC * NS
C = 128
NCHUNK = E // C

BN = 1000
BN1 = 1024
NP = 10240
BE = 2048

SLAB = 624
PSH = 30720
PSLAB = 1920

F32 = jnp.float32


def _silu(x):
    return x * jax.nn.sigmoid(x)


def _scmesh():
    return plsc.VectorSubcoreMesh(core_axis_name="c", subcore_axis_name="s",
                                  num_cores=NC, num_subcores=NS)


def _table_body(feat_ref, pos_ref, ws_ref, wr_ref, be1_ref, wrad_ref,
                ts_ref, tr_ref):
    f = feat_ref[...]
    p = pos_ref[...]
    n2 = jnp.sum(p * p, axis=1, keepdims=True) * wrad_ref[...]
    ts_ref[...] = jnp.dot(f, ws_ref[...], preferred_element_type=F32) + n2
    tr_ref[...] = (jnp.dot(f, wr_ref[...], preferred_element_type=F32)
                   + n2 + be1_ref[...])


def _build_tables(feat, pos, w_s, w_r, b_e1, w_rad):
    return pl.pallas_call(
        _table_body,
        grid=(N // BN,),
        in_specs=[
            pl.BlockSpec((BN, D), lambda i: (i, 0)),
            pl.BlockSpec((BN, 3), lambda i: (i, 0)),
            pl.BlockSpec((D, D), lambda i: (0, 0)),
            pl.BlockSpec((D, D), lambda i: (0, 0)),
            pl.BlockSpec((1, D), lambda i: (0, 0)),
            pl.BlockSpec((1, D), lambda i: (0, 0)),
        ],
        out_specs=[
            pl.BlockSpec((BN, D), lambda i: (i, 0)),
            pl.BlockSpec((BN, D), lambda i: (i, 0)),
        ],
        out_shape=[
            jax.ShapeDtypeStruct((N, D), F32),
            jax.ShapeDtypeStruct((N, D), F32),
        ],
    )(feat, pos, w_s, w_r, b_e1, w_rad)


def _gather_body(ts_hbm, tr_hbm, s_hbm, r_hbm, px_hbm, py_hbm, pz_hbm,
                 abs_out, abr_out, dot_out, cdp_out,
                 sidx0, ridx0, rows_s0, rows_r0,
                 psx0, psy0, psz0, prx0, pry0, prz0, smallb0,
                 sidx1, ridx1, rows_s1, rows_r1,
                 psx1, psy1, psz1, prx1, pry1, prz1, smallb1,
                 bounce, shx, shy, shz,
                 sem_s0, sem_r0, sem_p0, sem_s1, sem_r1, sem_p1):
    cid = lax.axis_index("c")
    sid = lax.axis_index("s")
    wid = sid * NC + cid
    nt = (NCHUNK - wid + NW - 1) // NW
    sets = [
        (sidx0, ridx0, rows_s0, rows_r0, psx0, psy0, psz0, prx0, pry0, prz0,
         smallb0, sem_s0, sem_r0, sem_p0),
        (sidx1, ridx1, rows_s1, rows_r1, psx1, psy1, psz1, prx1, pry1, prz1,
         smallb1, sem_s1, sem_r1, sem_p1),
    ]

    off = pl.multiple_of(sid * SLAB, 8)
    for plane_hbm, sh in ((px_hbm, shx), (py_hbm, shy), (pz_hbm, shz)):
        pltpu.sync_copy(plane_hbm.at[pl.ds(off, SLAB)], bounce)
        pltpu.sync_copy(bounce, sh.at[pl.ds(off, SLAB)])

    @pl.when(sid == NS - 1)
    def _():
        t0 = pl.multiple_of(NS * SLAB, 8)
        rem = N - NS * SLAB
        for plane_hbm, sh in ((px_hbm, shx), (py_hbm, shy), (pz_hbm, shz)):
            pltpu.sync_copy(plane_hbm.at[pl.ds(t0, rem)],
                            bounce.at[pl.ds(0, rem)])
            pltpu.sync_copy(bounce.at[pl.ds(0, rem)], sh.at[pl.ds(t0, rem)])

    plsc.subcore_barrier()

    def start(st, c):
        (sidx, ridx, rows_s, rows_r, psx, psy, psz, prx, pry, prz,
         _sm, sem_s, sem_r, sem_p) = st
        base = pl.multiple_of(c * C, C)
        pltpu.sync_copy(s_hbm.at[pl.ds(base, C)], sidx)
        pltpu.sync_copy(r_hbm.at[pl.ds(base, C)], ridx)
        pltpu.async_copy(ts_hbm.at[sidx], rows_s, sem_s)
        pltpu.async_copy(tr_hbm.at[ridx], rows_r, sem_r)
        pltpu.async_copy(shx.at[sidx], psx, sem_p)
        pltpu.async_copy(shy.at[sidx], psy, sem_p)
        pltpu.async_copy(shz.at[sidx], psz, sem_p)
        pltpu.async_copy(shx.at[ridx], prx, sem_p)
        pltpu.async_copy(shy.at[ridx], pry, sem_p)
        pltpu.async_copy(shz.at[ridx], prz, sem_p)

    def process(st, c):
        (sidx, ridx, rows_s, rows_r, psx, psy, psz, prx, pry, prz,
         smallb, sem_s, sem_r, sem_p) = st
        base = pl.multiple_of(c * C, C)
        for buf in (psx, psy, psz, prx, pry, prz):
            pltpu.make_async_copy(px_hbm.at[sidx], buf, sem_p).wait()

        for g in range(C // 16):
            sl = pl.ds(g * 16, 16)
            sx, sy, sz = psx[sl], psy[sl], psz[sl]
            rx, ry, rz = prx[sl], pry[sl], prz[sl]
            smallb[sl] = sx * rx + sy * ry + sz * rz
            smallb[pl.ds(C + g * 16, 16)] = sx - rx
            smallb[pl.ds(2 * C + g * 16, 16)] = sy - ry
            smallb[pl.ds(3 * C + g * 16, 16)] = sz - rz

        pltpu.sync_copy(smallb.at[pl.ds(0, C)], dot_out.at[pl.ds(base, C)])
        cbase = pl.multiple_of(c * (3 * C), C)
        pltpu.sync_copy(smallb.at[pl.ds(C, 3 * C)],
                        cdp_out.at[pl.ds(cbase, 3 * C)])
        pltpu.make_async_copy(ts_hbm.at[sidx], rows_s, sem_s).wait()
        pltpu.make_async_copy(tr_hbm.at[ridx], rows_r, sem_r).wait()
        pltpu.sync_copy(rows_s, abs_out.at[pl.ds(base, C)])
        pltpu.sync_copy(rows_r, abr_out.at[pl.ds(base, C)])

    start(sets[0], wid)

    def pair(k2, _):
        for half in (0, 1):
            k = 2 * k2 + half

            @pl.when(k + 1 < nt)
            def _():
                start(sets[1 - half], wid + (k + 1) * NW)

            @pl.when(k < nt)
            def _():
                process(sets[half], wid + k * NW)
        return 0

    lax.fori_loop(0, 40, pair, 0)


def _gather_edges(ts, tr, senders, receivers, px, py, pz):
    fn = functools.partial(
        pl.kernel,
        out_type=[
            jax.ShapeDtypeStruct((EP, D), F32),
            jax.ShapeDtypeStruct((EP, D), F32),
            jax.ShapeDtypeStruct((EP,), F32),
            jax.ShapeDtypeStruct((3 * E,), F32),
        ],
        mesh=_scmesh(),
        scratch_types=(
            2 * ([pltpu.VMEM((C,), jnp.int32)] * 2
                 + [pltpu.VMEM((C, D), F32)] * 2
                 + [pltpu.VMEM((C,), F32)] * 6
                 + [pltpu.VMEM((4 * C,), F32)])
            + [pltpu.VMEM((SLAB,), F32)]
            + [pltpu.VMEM_SHARED((N,), F32)] * 3
            + [pltpu.SemaphoreType.DMA] * 6
        ),
    )(_gather_body)
    return fn(ts, tr, senders, receivers, px, py, pz)


def _edge_body(abs_ref, abr_ref, attr_ref, dot_ref, wa_ref, wrad2_ref,
               we2_ref, be2_ref, wp1_ref, bp1_ref, wp2_ref,
               msg_ref, pc_ref):
    dcol = dot_ref[...].reshape((BE, 1))
    pre1 = (abs_ref[...] + abr_ref[...]
            + jnp.dot(attr_ref[...], wa_ref[...], preferred_element_type=F32)
            + dcol * wrad2_ref[...])
    h = _silu(pre1)
    m = _silu(jnp.dot(h, we2_ref[...], preferred_element_type=F32) + be2_ref[...])
    msg_ref[...] = m
    h2 = _silu(jnp.dot(m, wp1_ref[...], preferred_element_type=F32) + bp1_ref[...])
    pc_ref[...] = jnp.sum(h2 * wp2_ref[...], axis=1)


def _edge_mlp(ab_s, ab_r, attr, dot, w_a, w_rad2, w_e2, b_e2, w_p1, b_p1, w_p2row):
    return pl.pallas_call(
        _edge_body,
        grid=(EP // BE,),
        in_specs=[
            pl.BlockSpec((BE, D), lambda i: (i, 0)),
            pl.BlockSpec((BE, D), lambda i: (i, 0)),
            pl.BlockSpec((BE, DE), lambda i: (i, 0)),
            pl.BlockSpec((BE,), lambda i: (i,)),
            pl.BlockSpec((DE, D), lambda i: (0, 0)),
            pl.BlockSpec((1, D), lambda i: (0, 0)),
            pl.BlockSpec((D, D), lambda i: (0, 0)),
            pl.BlockSpec((1, D), lambda i: (0, 0)),
            pl.BlockSpec((D, D), lambda i: (0, 0)),
            pl.BlockSpec((1, D), lambda i: (0, 0)),
            pl.BlockSpec((1, D), lambda i: (0, 0)),
        ],
        out_specs=[
            pl.BlockSpec((BE, D), lambda i: (i, 0)),
            pl.BlockSpec((BE,), lambda i: (i,)),
        ],
        out_shape=[
            jax.ShapeDtypeStruct((EP, D), F32),
            jax.ShapeDtypeStruct((EP,), F32),
        ],
    )(ab_s, ab_r, attr, dot, w_a, w_rad2, w_e2, b_e2, w_p1, b_p1, w_p2row)


def _scatter_body(msg_hbm, pc_hbm, cdp_hbm, r_hbm, s_hbm,
                  agg_out, pos_out,
                  ridx, sidx, mbuf, pcb, cdb, tbx, tby, tbz,
                  ridx1, sidx1, mbuf1, pcb1, cdb1, tbx1, tby1, tbz1,
                  bounce, agg_sh, pos_shx, pos_shy, pos_shz,
                  sem_rd0, sem_sc0, sem_rd1, sem_sc1):
    cid = lax.axis_index("c")
    sid = lax.axis_index("s")
    wid = sid * NC + cid
    z16 = jnp.zeros((16,), F32)
    planes = (pos_shx, pos_shy, pos_shz)

    def zrow(j, _):
        for q in range(D // 16):
            mbuf[j, pl.ds(q * 16, 16)] = z16
        return 0

    lax.fori_loop(0, C, zrow, 0)
    for g in range(SLAB // 16):
        bounce[pl.ds(g * 16, 16)] = z16

    slab0 = pl.multiple_of(sid * SLAB, 8)

    def zslab(j, _):
        off = pl.multiple_of(slab0 + j * C, 8)
        pltpu.sync_copy(mbuf, agg_sh.at[pl.ds(off, C)])
        return 0

    lax.fori_loop(0, SLAB // C, zslab, 0)
    off = pl.multiple_of(slab0 + (SLAB // C) * C, 8)
    pltpu.sync_copy(mbuf.at[pl.ds(0, SLAB % C)], agg_sh.at[pl.ds(off, SLAB % C)])
    for plane in planes:
        pltpu.sync_copy(bounce, plane.at[pl.ds(slab0, SLAB)])

    @pl.when(sid == NS - 1)
    def _():
        tail = pl.multiple_of(NS * SLAB, 8)
        rem = N - NS * SLAB
        pltpu.sync_copy(mbuf.at[pl.ds(0, rem)], agg_sh.at[pl.ds(tail, rem)])
        for plane in planes:
            pltpu.sync_copy(bounce.at[pl.ds(0, rem)], plane.at[pl.ds(tail, rem)])

    plsc.subcore_barrier()

    nt = (NCHUNK - wid + NW - 1) // NW
    sets = [
        (ridx, sidx, mbuf, pcb, cdb, tbx, tby, tbz, sem_rd0, sem_sc0),
        (ridx1, sidx1, mbuf1, pcb1, cdb1, tbx1, tby1, tbz1, sem_rd1, sem_sc1),
    ]

    def start(st, c, k):
        (rx, sx, mb, pb, cb, _tx, _ty, _tz, sem_rd, sem_sc) = st
        base = pl.multiple_of(c * C, C)
        cbase = pl.multiple_of(c * (3 * C), C)

        @pl.when(k >= 2)
        def _():
            pltpu.make_async_copy(mb, agg_sh.at[rx], sem_sc).wait()
            pltpu.make_async_copy(_tx, pos_shx.at[sx], sem_sc).wait()
            pltpu.make_async_copy(_ty, pos_shy.at[sx], sem_sc).wait()
            pltpu.make_async_copy(_tz, pos_shz.at[sx], sem_sc).wait()

        pltpu.async_copy(r_hbm.at[pl.ds(base, C)], rx, sem_rd)
        pltpu.async_copy(msg_hbm.at[pl.ds(base, C)], mb, sem_rd)
        pltpu.async_copy(s_hbm.at[pl.ds(base, C)], sx, sem_rd)
        pltpu.async_copy(pc_hbm.at[pl.ds(base, C)], pb, sem_rd)
        pltpu.async_copy(cdp_hbm.at[pl.ds(cbase, 3 * C)], cb, sem_rd)

    def process(st, c):
        (rx, sx, mb, pb, cb, tx, ty, tz, sem_rd, sem_sc) = st
        base = pl.multiple_of(c * C, C)
        cbase = pl.multiple_of(c * (3 * C), C)
        pltpu.make_async_copy(r_hbm.at[pl.ds(base, C)], rx, sem_rd).wait()
        pltpu.make_async_copy(msg_hbm.at[pl.ds(base, C)], mb, sem_rd).wait()
        pltpu.make_async_copy(s_hbm.at[pl.ds(base, C)], sx, sem_rd).wait()
        pltpu.make_async_copy(pc_hbm.at[pl.ds(base, C)], pb, sem_rd).wait()
        pltpu.make_async_copy(cdp_hbm.at[pl.ds(cbase, 3 * C)], cb, sem_rd).wait()
        for g in range(C // 16):
            sl = pl.ds(g * 16, 16)
            pcv = pb[sl]
            tx[sl] = jnp.clip(cb[sl] * pcv, -100.0, 100.0)
            ty[sl] = jnp.clip(cb[pl.ds(C + g * 16, 16)] * pcv, -100.0, 100.0)
            tz[sl] = jnp.clip(cb[pl.ds(2 * C + g * 16, 16)] * pcv, -100.0, 100.0)
        pltpu.async_copy(mb, agg_sh.at[rx], sem_sc, add=True)
        pltpu.async_copy(tx, pos_shx.at[sx], sem_sc, add=True)
        pltpu.async_copy(ty, pos_shy.at[sx], sem_sc, add=True)
        pltpu.async_copy(tz, pos_shz.at[sx], sem_sc, add=True)

    start(sets[0], wid, 0)

    def pair(k2, _):
        for half in (0, 1):
            k = 2 * k2 + half

            @pl.when(k + 1 < nt)
            def _():
                start(sets[1 - half], wid + (k + 1) * NW, k + 1)

            @pl.when(k < nt)
            def _():
                process(sets[half], wid + k * NW)
        return 0

    lax.fori_loop(0, 40, pair, 0)

    for st in sets:
        (rx, sx, mb, _pb, _cb, tx, ty, tz, _sem_rd, sem_sc) = st
        pltpu.make_async_copy(mb, agg_sh.at[rx], sem_sc).wait()
        pltpu.make_async_copy(tx, pos_shx.at[sx], sem_sc).wait()
        pltpu.make_async_copy(ty, pos_shy.at[sx], sem_sc).wait()
        pltpu.make_async_copy(tz, pos_shz.at[sx], sem_sc).wait()

    plsc.subcore_barrier()

    pltpu.sync_copy(agg_sh.at[pl.ds(slab0, SLAB)],
                    agg_out.at[cid, pl.ds(slab0, SLAB)])
    for comp, plane in enumerate(planes):
        dst = pl.multiple_of((cid * 3 + comp) * NP + slab0, 8)
        pltpu.sync_copy(plane.at[pl.ds(slab0, SLAB)], bounce)
        pltpu.sync_copy(bounce, pos_out.at[pl.ds(dst, SLAB)])

    @pl.when(sid == NS - 1)
    def _():
        t0 = pl.multiple_of(NS * SLAB, 8)
        rem = N - NS * SLAB
        pltpu.sync_copy(agg_sh.at[pl.ds(t0, rem)],
                        agg_out.at[cid, pl.ds(t0, rem)])
        for comp, plane in enumerate(planes):
            dst = pl.multiple_of((cid * 3 + comp) * NP + t0, 8)
            pltpu.sync_copy(plane.at[pl.ds(t0, rem)], bounce.at[pl.ds(0, rem)])
            pltpu.sync_copy(bounce.at[pl.ds(0, rem)], pos_out.at[pl.ds(dst, rem)])


def _scatter_edges(msg, pc, cdp, receivers, senders):
    fn = functools.partial(
        pl.kernel,
        out_type=[
            jax.ShapeDtypeStruct((NC, N, D), F32),
            jax.ShapeDtypeStruct((NC * 3 * NP,), F32),
        ],
        mesh=_scmesh(),
        scratch_types=(
            2 * ([pltpu.VMEM((C,), jnp.int32)] * 2
                 + [pltpu.VMEM((C, D), F32)]
                 + [pltpu.VMEM((C,), F32)]
                 + [pltpu.VMEM((3 * C,), F32)]
                 + [pltpu.VMEM((C,), F32)] * 3)
            + [pltpu.VMEM((SLAB,), F32)]
            + [pltpu.VMEM_SHARED((N, D), F32)]
            + [pltpu.VMEM_SHARED((N,), F32)] * 3
            + [pltpu.SemaphoreType.DMA] * 4
        ),
    )(_scatter_body)
    return fn(msg, pc, cdp, receivers, senders)


def _node_body(feat_ref, aggp_ref, px_ref, py_ref, pz_ref,
               p0x_ref, p0y_ref, p0z_ref, p1x_ref, p1y_ref, p1z_ref,
               wn1a_ref, wn1b_ref, bn1_ref, wn2_ref, bn2_ref,
               fout_ref, pxo_ref, pyo_ref, pzo_ref):
    f = feat_ref[...]
    agg = aggp_ref[0] + aggp_ref[1]
    x = _silu(jnp.dot(f, wn1a_ref[...], preferred_element_type=F32)
              + jnp.dot(agg, wn1b_ref[...], preferred_element_type=F32)
              + bn1_ref[...])
    fout_ref[...] = f + jnp.dot(x, wn2_ref[...], preferred_element_type=F32) + bn2_ref[...]
    pxo_ref[...] = px_ref[...] + p0x_ref[...] + p1x_ref[...]
    pyo_ref[...] = py_ref[...] + p0y_ref[...] + p1y_ref[...]
    pzo_ref[...] = pz_ref[...] + p0z_ref[...] + p1z_ref[...]


def _node_update(feat, aggp, posacc, pxp, pyp, pzp,
                 w_n1a, w_n1b, b_n1, w_n2, b_n2):
    nb = NP // BN1
    acc_spec = lambda p: pl.BlockSpec((BN1,), lambda i, p=p: (i + p * nb,))
    return pl.pallas_call(
        _node_body,
        grid=(N // BN,),
        in_specs=[
            pl.BlockSpec((BN, D), lambda i: (i, 0)),
            pl.BlockSpec((NC, BN, D), lambda i: (0, i, 0)),
            pl.BlockSpec((BN1,), lambda i: (i,)),
            pl.BlockSpec((BN1,), lambda i: (i,)),
            pl.BlockSpec((BN1,), lambda i: (i,)),
            acc_spec(0), acc_spec(1), acc_spec(2),
            acc_spec(3), acc_spec(4), acc_spec(5),
            pl.BlockSpec((D, D), lambda i: (0, 0)),
            pl.BlockSpec((D, D), lambda i: (0, 0)),
            pl.BlockSpec((1, D), lambda i: (0, 0)),
            pl.BlockSpec((D, D), lambda i: (0, 0)),
            pl.BlockSpec((1, D), lambda i: (0, 0)),
        ],
        out_specs=[
            pl.BlockSpec((BN, D), lambda i: (i, 0)),
            pl.BlockSpec((BN1,), lambda i: (i,)),
            pl.BlockSpec((BN1,), lambda i: (i,)),
            pl.BlockSpec((BN1,), lambda i: (i,)),
        ],
        out_shape=[
            jax.ShapeDtypeStruct((N, D), F32),
            jax.ShapeDtypeStruct((NP,), F32),
            jax.ShapeDtypeStruct((NP,), F32),
            jax.ShapeDtypeStruct((NP,), F32),
        ],
    )(feat, aggp, pxp, pyp, pzp,
      posacc, posacc, posacc, posacc, posacc, posacc,
      w_n1a, w_n1b, b_n1, w_n2, b_n2)


def kernel(feat, pos, senders, receivers, edge_attribute,
           W_e1, b_e1, W_e2, b_e2, W_n1, b_n1, W_n2, b_n2, W_p1, b_p1, W_p2):
    senders = senders.astype(jnp.int32)
    receivers = receivers.astype(jnp.int32)

    w_s = W_e1[:D]
    w_r = W_e1[D:2 * D]
    w_rad = W_e1[2 * D:2 * D + 1]
    w_rad2 = -2.0 * w_rad
    w_a = W_e1[2 * D + 1:]

    px = pos[:, 0]
    py = pos[:, 1]
    pz = pos[:, 2]
    pxp = jnp.pad(px, (0, NP - N))
    pyp = jnp.pad(py, (0, NP - N))
    pzp = jnp.pad(pz, (0, NP - N))
    attr_p = jnp.pad(edge_attribute, ((0, EP - E), (0, 0)))

    ts, tr = _build_tables(feat, pos, w_s, w_r, b_e1.reshape(1, D), w_rad)
    ab_s, ab_r, dot, cdp = _gather_edges(ts, tr, senders, receivers, px, py, pz)
    msg, pc = _edge_mlp(ab_s, ab_r, attr_p, dot, w_a, w_rad2,
                        W_e2, b_e2.reshape(1, D), W_p1, b_p1.reshape(1, D),
                        W_p2.reshape(1, D))
    aggp, posacc = _scatter_edges(msg, pc, cdp, receivers, senders)
    feat_new, pxo, pyo, pzo = _node_update(feat, aggp, posacc, pxp, pyp, pzp,
                                           W_n1[:D], W_n1[D:],
                                           b_n1.reshape(1, D), W_n2,
                                           b_n2.reshape(1, D))
    pos_new = jnp.stack([pxo[:N], pyo[:N], pzo[:N]], axis=1)
    return feat_new, pos_new
